# baseline (speedup 1.0000x reference)
import functools

import jax
import jax.numpy as jnp
from jax import lax
from jax.experimental import pallas as pl
from jax.experimental.pallas import tpu as pltpu
from jax.experimental.pallas import tpu_sc as plsc

NC = 2
NS = 16
L = 16
NW = NC * NS

B = 16384
D = 64
BPW = B // NW
NCHUNK = 4
CHUNK = BPW // NCHUNK

_mesh = plsc.VectorSubcoreMesh(
    core_axis_name="c", subcore_axis_name="s", num_cores=NC, num_subcores=NS
)


@functools.partial(
    pl.kernel,
    out_type=jax.ShapeDtypeStruct((NW, BPW), jnp.float32),
    mesh=_mesh,
    compiler_params=pltpu.CompilerParams(use_tc_tiling_on_sc=False),
    scratch_types=[
        pltpu.VMEM((NCHUNK, CHUNK), jnp.int32),
        pltpu.VMEM((NCHUNK, CHUNK), jnp.int32),
        pltpu.VMEM((BPW, D), jnp.float32),
        pltpu.VMEM((BPW, D), jnp.float32),
        pltpu.VMEM((BPW,), jnp.float32),
        pltpu.VMEM((BPW,), jnp.float32),
        pltpu.VMEM((BPW,), jnp.float32),
        pltpu.SMEM((BPW,), jnp.float32),
        pltpu.SemaphoreType.DMA,
    ],
)
def _scores_kernel(uids_hbm, iids_hbm, ue_hbm, ie_hbm, ub_hbm, ib_hbm,
                   out_hbm, uidx_v, iidx_v, ue_v, ie_v,
                   ub_v, ib_v, out_v, dot_s, sem):
    wid = lax.axis_index("s") * NC + lax.axis_index("c")

    pltpu.sync_copy(uids_hbm.at[wid], uidx_v)
    pltpu.sync_copy(iids_hbm.at[wid], iidx_v)

    copies = []
    for c in range(NCHUNK):
        sl = pl.ds(c * CHUNK, CHUNK)
        copies.append(pltpu.async_copy(ue_hbm.at[uidx_v.at[c]], ue_v.at[sl], sem))
        copies.append(pltpu.async_copy(ie_hbm.at[iidx_v.at[c]], ie_v.at[sl], sem))
        copies.append(pltpu.async_copy(ub_hbm.at[uidx_v.at[c]], ub_v.at[sl], sem))
        copies.append(pltpu.async_copy(ib_hbm.at[iidx_v.at[c]], ib_v.at[sl], sem))
    for cp in copies:
        cp.wait()

    lane = lax.iota(jnp.int32, L)

    def hsum(v):
        for dist in (8, 4, 2, 1):
            v = v + v.at[lane ^ dist].get(mode="promise_in_bounds")
        return v

    def body(g, carry):
        row0 = g * L
        res = jnp.zeros((L,), jnp.float32)
        for k in range(L):
            r = row0 + k
            acc = ue_v[r, pl.ds(0, L)] * ie_v[r, pl.ds(0, L)]
            for c in range(1, D // L):
                acc = acc + ue_v[r, pl.ds(c * L, L)] * ie_v[r, pl.ds(c * L, L)]
            res = jnp.where(lane == k, hsum(acc), res)
        sl = pl.ds(row0, L)
        out_v[sl] = res + ub_v[sl] + ib_v[sl]
        return carry

    lax.fori_loop(0, BPW // L, body, 0)

    pltpu.sync_copy(out_v, out_hbm.at[wid])


def kernel(user_ids, item_ids, user_embed, item_embed, user_bias, item_bias):
    uids = user_ids.astype(jnp.int32).reshape(NW, NCHUNK, CHUNK)
    iids = item_ids.astype(jnp.int32).reshape(NW, NCHUNK, CHUNK)
    out = _scores_kernel(uids, iids, user_embed, item_embed,
                         user_bias.reshape(-1), item_bias.reshape(-1))
    return out.reshape(B)


# speedup vs baseline: 1.0952x; 1.0952x over previous
import functools

import jax
import jax.numpy as jnp
from jax import lax
from jax.experimental import pallas as pl
from jax.experimental.pallas import tpu as pltpu
from jax.experimental.pallas import tpu_sc as plsc

NC = 2
NS = 16
L = 16
NW = NC * NS

B = 16384
D = 64
BPW = B // NW
NCHUNK = 4
CHUNK = BPW // NCHUNK
NGROUP = BPW // L


_K1_COLS = 1024
_HALF_BLOCKS = 489
HALF = _K1_COLS * _HALF_BLOCKS
_LAST_BLOCK = 976


def _relayout_body(lo_ref, hi_ref, out_ref):
    out_ref[...] = jnp.concatenate([lo_ref[...].T, hi_ref[...].T], axis=1)


def _relayout(table_t):
    return pl.pallas_call(
        _relayout_body,
        grid=(_HALF_BLOCKS,),
        in_specs=[
            pl.BlockSpec((D, _K1_COLS), lambda i: (0, i)),
            pl.BlockSpec(
                (D, _K1_COLS),
                lambda i: (0, jnp.minimum(i + _HALF_BLOCKS, _LAST_BLOCK)),
            ),
        ],
        out_specs=pl.BlockSpec((_K1_COLS, 128), lambda i: (i, 0)),
        out_shape=jax.ShapeDtypeStruct((HALF, 128), jnp.float32),
    )(table_t, table_t)


_mesh = plsc.VectorSubcoreMesh(
    core_axis_name="c", subcore_axis_name="s", num_cores=NC, num_subcores=NS
)


@functools.partial(
    pl.kernel,
    out_type=jax.ShapeDtypeStruct((NW, BPW), jnp.float32),
    mesh=_mesh,
    scratch_types=[
        pltpu.VMEM((NCHUNK, CHUNK), jnp.int32),
        pltpu.VMEM((NCHUNK, CHUNK), jnp.int32),
        pltpu.VMEM((NCHUNK, CHUNK), jnp.int32),
        pltpu.VMEM((NCHUNK, CHUNK), jnp.int32),
        pltpu.VMEM((NGROUP, L), jnp.int32),
        pltpu.VMEM((NGROUP, L), jnp.int32),
        pltpu.VMEM((CHUNK, 128), jnp.float32),
        pltpu.VMEM((CHUNK, 128), jnp.float32),
        pltpu.VMEM((CHUNK, 128), jnp.float32),
        pltpu.VMEM((CHUNK, 128), jnp.float32),
        pltpu.VMEM((BPW,), jnp.float32),
        pltpu.VMEM((BPW,), jnp.float32),
        pltpu.VMEM((BPW,), jnp.float32),
        pltpu.SemaphoreType.DMA,
        pltpu.SemaphoreType.DMA,
        pltpu.SemaphoreType.DMA,
    ],
)
def _scores_kernel(uid4_hbm, iid4_hbm, uhid_hbm, ihid_hbm, uidr_hbm, iidr_hbm,
                   uer_hbm, ier_hbm, ub_hbm, ib_hbm,
                   out_hbm, uid_v, iid_v, uhid_v, ihid_v, uidr_v, iidr_v,
                   ue_a, ue_b, ie_a, ie_b, ub_v, ib_v, out_v,
                   sem_a, sem_b, sem):
    wid = lax.axis_index("s") * NC + lax.axis_index("c")

    pltpu.sync_copy(uid4_hbm.at[wid], uid_v)
    pltpu.sync_copy(iid4_hbm.at[wid], iid_v)
    pltpu.sync_copy(uhid_hbm.at[wid], uhid_v)
    pltpu.sync_copy(ihid_hbm.at[wid], ihid_v)
    pltpu.sync_copy(uidr_hbm.at[wid], uidr_v)
    pltpu.sync_copy(iidr_hbm.at[wid], iidr_v)

    bias_copies = []
    for c in range(NCHUNK):
        sl = pl.ds(c * CHUNK, CHUNK)
        bias_copies.append(
            pltpu.async_copy(ub_hbm.at[uid_v.at[c]], ub_v.at[sl], sem))
        bias_copies.append(
            pltpu.async_copy(ib_hbm.at[iid_v.at[c]], ib_v.at[sl], sem))

    bufs = [(ue_a, ie_a, sem_a), (ue_b, ie_b, sem_b)]

    def fire(c):
        ue, ie, s = bufs[c % 2]
        return (pltpu.async_copy(uer_hbm.at[uhid_v.at[c]], ue, s),
                pltpu.async_copy(ier_hbm.at[ihid_v.at[c]], ie, s))

    lane = lax.iota(jnp.int32, L)
    halfn = jnp.full((L,), HALF, jnp.int32)
    c64 = jnp.full((L,), 64, jnp.int32)
    c0 = jnp.zeros((L,), jnp.int32)

    def hsum(v):
        for dist in (8, 4, 2, 1):
            v = v + v.at[lane ^ dist].get(mode="promise_in_bounds")
        return v

    gpc = CHUNK // L

    cur = fire(0)
    for c in range(NCHUNK):
        nxt = fire(c + 1) if c + 1 < NCHUNK else None
        cur[0].wait()
        cur[1].wait()
        if c == 0:
            for cp in bias_copies:
                cp.wait()
        ue_v, ie_v, _ = bufs[c % 2]

        def body(gi, carry, c=c, ue_v=ue_v, ie_v=ie_v):
            g = c * gpc + gi
            row0 = gi * L
            hvu = jnp.where(uidr_v[g] >= halfn, c64, c0)
            hvi = jnp.where(iidr_v[g] >= halfn, c64, c0)
            res = jnp.zeros((L,), jnp.float32)
            for k in range(L):
                hu = hvu[k]
                hi = hvi[k]
                r = row0 + k
                acc = (ue_v[r, pl.ds(hu, L)] * ie_v[r, pl.ds(hi, L)])
                for cc in range(1, D // L):
                    acc = acc + (ue_v[r, pl.ds(hu + cc * L, L)]
                                 * ie_v[r, pl.ds(hi + cc * L, L)])
                res = jnp.where(lane == k, hsum(acc), res)
            sl = pl.ds(g * L, L)
            out_v[sl] = res + ub_v[sl] + ib_v[sl]
            return carry

        lax.fori_loop(0, gpc, body, 0)
        cur = nxt

    pltpu.sync_copy(out_v, out_hbm.at[wid])


def kernel(user_ids, item_ids, user_embed, item_embed, user_bias, item_bias):
    uids = user_ids.astype(jnp.int32)
    iids = item_ids.astype(jnp.int32)
    uer = _relayout(user_embed.T)
    ier = _relayout(item_embed.T)
    out = _scores_kernel(
        uids.reshape(NW, NCHUNK, CHUNK),
        iids.reshape(NW, NCHUNK, CHUNK),
        (uids % HALF).reshape(NW, NCHUNK, CHUNK),
        (iids % HALF).reshape(NW, NCHUNK, CHUNK),
        uids.reshape(NW, NGROUP, L),
        iids.reshape(NW, NGROUP, L),
        uer,
        ier,
        user_bias.reshape(-1),
        item_bias.reshape(-1),
    )
    return out.reshape(B)


# speedup vs baseline: 2.1710x; 1.9822x over previous
import functools

import jax
import jax.numpy as jnp
from jax import lax
from jax.experimental import pallas as pl
from jax.experimental.pallas import tpu as pltpu
from jax.experimental.pallas import tpu_sc as plsc

NC = 2
NS = 16
L = 16
NW = NC * NS

B = 16384
D = 64
BPW = B // NW
NCHUNK = 4
CHUNK = BPW // NCHUNK
NGROUP = BPW // L


_K1_COLS = 4096
_HALF_BLOCKS = 123
HALF = _K1_COLS * _HALF_BLOCKS
_LAST_BLOCK = 244


def _relayout_body(lo_ref, hi_ref, out_ref):
    stacked = jnp.concatenate([lo_ref[...], hi_ref[...]], axis=0)
    ii = lax.broadcasted_iota(jnp.int32, (128, 128), 0)
    jj = lax.broadcasted_iota(jnp.int32, (128, 128), 1)
    eye = (ii == jj).astype(jnp.float32)
    out_ref[...] = lax.dot_general(
        stacked, eye, (((0,), (0,)), ((), ())),
        preferred_element_type=jnp.float32,
    )


def _relayout(table_t):
    return pl.pallas_call(
        _relayout_body,
        grid=(_HALF_BLOCKS,),
        in_specs=[
            pl.BlockSpec((D, _K1_COLS), lambda i: (0, i)),
            pl.BlockSpec(
                (D, _K1_COLS),
                lambda i: (0, jnp.minimum(i + _HALF_BLOCKS, _LAST_BLOCK)),
            ),
        ],
        out_specs=pl.BlockSpec((_K1_COLS, 128), lambda i: (i, 0)),
        out_shape=jax.ShapeDtypeStruct((HALF, 128), jnp.float32),
        compiler_params=pltpu.CompilerParams(
            fuse_transposed_lhs_in_matmul=True),
    )(table_t, table_t)


_mesh = plsc.VectorSubcoreMesh(
    core_axis_name="c", subcore_axis_name="s", num_cores=NC, num_subcores=NS
)


@functools.partial(
    pl.kernel,
    out_type=jax.ShapeDtypeStruct((NW, BPW), jnp.float32),
    mesh=_mesh,
    scratch_types=[
        pltpu.VMEM((NCHUNK, CHUNK), jnp.int32),
        pltpu.VMEM((NCHUNK, CHUNK), jnp.int32),
        pltpu.VMEM((NCHUNK, CHUNK), jnp.int32),
        pltpu.VMEM((NCHUNK, CHUNK), jnp.int32),
        pltpu.VMEM((NGROUP, L), jnp.int32),
        pltpu.VMEM((NGROUP, L), jnp.int32),
        pltpu.VMEM((CHUNK, 128), jnp.float32),
        pltpu.VMEM((CHUNK, 128), jnp.float32),
        pltpu.VMEM((CHUNK, 128), jnp.float32),
        pltpu.VMEM((CHUNK, 128), jnp.float32),
        pltpu.VMEM((BPW,), jnp.float32),
        pltpu.VMEM((BPW,), jnp.float32),
        pltpu.VMEM((BPW,), jnp.float32),
        pltpu.SemaphoreType.DMA,
        pltpu.SemaphoreType.DMA,
        pltpu.SemaphoreType.DMA,
    ],
)
def _scores_kernel(uid4_hbm, iid4_hbm, uhid_hbm, ihid_hbm, uidr_hbm, iidr_hbm,
                   uer_hbm, ier_hbm, ub_hbm, ib_hbm,
                   out_hbm, uid_v, iid_v, uhid_v, ihid_v, uidr_v, iidr_v,
                   ue_a, ue_b, ie_a, ie_b, ub_v, ib_v, out_v,
                   sem_a, sem_b, sem):
    wid = lax.axis_index("s") * NC + lax.axis_index("c")

    pltpu.sync_copy(uid4_hbm.at[wid], uid_v)
    pltpu.sync_copy(iid4_hbm.at[wid], iid_v)
    pltpu.sync_copy(uhid_hbm.at[wid], uhid_v)
    pltpu.sync_copy(ihid_hbm.at[wid], ihid_v)
    pltpu.sync_copy(uidr_hbm.at[wid], uidr_v)
    pltpu.sync_copy(iidr_hbm.at[wid], iidr_v)

    bias_copies = []
    for c in range(NCHUNK):
        sl = pl.ds(c * CHUNK, CHUNK)
        bias_copies.append(
            pltpu.async_copy(ub_hbm.at[uid_v.at[c]], ub_v.at[sl], sem))
        bias_copies.append(
            pltpu.async_copy(ib_hbm.at[iid_v.at[c]], ib_v.at[sl], sem))

    bufs = [(ue_a, ie_a, sem_a), (ue_b, ie_b, sem_b)]

    def fire(c):
        ue, ie, s = bufs[c % 2]
        return (pltpu.async_copy(uer_hbm.at[uhid_v.at[c]], ue, s),
                pltpu.async_copy(ier_hbm.at[ihid_v.at[c]], ie, s))

    lane = lax.iota(jnp.int32, L)
    halfn = jnp.full((L,), HALF, jnp.int32)
    c64 = jnp.full((L,), 64, jnp.int32)
    c0 = jnp.zeros((L,), jnp.int32)

    def hsum(v):
        for dist in (8, 4, 2, 1):
            v = v + v.at[lane ^ dist].get(mode="promise_in_bounds")
        return v

    gpc = CHUNK // L

    cur = fire(0)
    for c in range(NCHUNK):
        nxt = fire(c + 1) if c + 1 < NCHUNK else None
        cur[0].wait()
        cur[1].wait()
        if c == 0:
            for cp in bias_copies:
                cp.wait()
        ue_v, ie_v, _ = bufs[c % 2]

        def body(gi, carry, c=c, ue_v=ue_v, ie_v=ie_v):
            g = c * gpc + gi
            row0 = gi * L
            hvu = jnp.where(uidr_v[g] >= halfn, c64, c0)
            hvi = jnp.where(iidr_v[g] >= halfn, c64, c0)
            res = jnp.zeros((L,), jnp.float32)
            for k in range(L):
                hu = hvu[k]
                hi = hvi[k]
                r = row0 + k
                acc = (ue_v[r, pl.ds(hu, L)] * ie_v[r, pl.ds(hi, L)])
                for cc in range(1, D // L):
                    acc = acc + (ue_v[r, pl.ds(hu + cc * L, L)]
                                 * ie_v[r, pl.ds(hi + cc * L, L)])
                res = jnp.where(lane == k, hsum(acc), res)
            sl = pl.ds(g * L, L)
            out_v[sl] = res + ub_v[sl] + ib_v[sl]
            return carry

        lax.fori_loop(0, gpc, body, 0)
        cur = nxt

    pltpu.sync_copy(out_v, out_hbm.at[wid])


def kernel(user_ids, item_ids, user_embed, item_embed, user_bias, item_bias):
    uids = user_ids.astype(jnp.int32)
    iids = item_ids.astype(jnp.int32)
    uer = _relayout(user_embed.T)
    ier = _relayout(item_embed.T)
    out = _scores_kernel(
        uids.reshape(NW, NCHUNK, CHUNK),
        iids.reshape(NW, NCHUNK, CHUNK),
        (uids % HALF).reshape(NW, NCHUNK, CHUNK),
        (iids % HALF).reshape(NW, NCHUNK, CHUNK),
        uids.reshape(NW, NGROUP, L),
        iids.reshape(NW, NGROUP, L),
        uer,
        ier,
        user_bias.reshape(-1),
        item_bias.reshape(-1),
    )
    return out.reshape(B)
